# Initial kernel scaffold; baseline (speedup 1.0000x reference)
import jax
import jax.numpy as jnp
from jax.experimental import pallas as pl


def kernel(x, edge_index, W0, W1, b1, Wm, bm, W2, b2):
    raise NotImplementedError("write your pallas kernel here")


# speedup vs baseline: 6.2500x; 6.2500x over previous
import functools

import jax
import jax.numpy as jnp
from jax import lax
from jax.experimental import pallas as pl
from jax.experimental.pallas import tpu as pltpu
from jax.experimental.pallas import tpu_sc as plsc

N = 10000
E = 320000
NFEAT = 128
NHID = 128
NCLASS = 64

NC = 2
NS = 16
NW = NC * NS
E_PER_TILE = E // NW
CH = 80
NCH = E_PER_TILE // CH
ROWS_PER_TILE = 624
ROWS_REM = N - NS * ROWS_PER_TILE

BLK = 1000


def _make_agg(D):
    mesh = plsc.VectorSubcoreMesh(core_axis_name="c", subcore_axis_name="s")

    @functools.partial(
        pl.kernel,
        out_type=jax.ShapeDtypeStruct((NC, N, D), jnp.float32),
        mesh=mesh,
        scratch_types=[
            pltpu.VMEM((NCH, CH), jnp.int32),
            pltpu.VMEM((NCH, CH), jnp.int32),
            pltpu.VMEM((CH, D), jnp.float32),
            pltpu.VMEM_SHARED((N, D), jnp.float32),
            pltpu.SemaphoreType.DMA,
        ],
    )
    def agg(support, src3, dst3, zeros, out, src_v, dst_v, rows_v, acc, sem):
        c = lax.axis_index("c")
        s = lax.axis_index("s")
        wid = c * NS + s
        pltpu.sync_copy(src3.at[wid], src_v)
        pltpu.sync_copy(dst3.at[wid], dst_v)
        pltpu.sync_copy(zeros, acc.at[pl.ds(s * ROWS_PER_TILE, ROWS_PER_TILE)])

        @pl.when(s == 0)
        def _():
            pltpu.sync_copy(zeros.at[pl.ds(0, ROWS_REM)],
                            acc.at[pl.ds(NS * ROWS_PER_TILE, ROWS_REM)])

        plsc.subcore_barrier()

        def body(i, carry):
            pltpu.async_copy(support.at[src_v.at[i]], rows_v, sem).wait()
            pltpu.sync_copy(rows_v, acc.at[dst_v.at[i]], add=True)
            return carry

        lax.fori_loop(0, NCH, body, 0)
        plsc.subcore_barrier()
        pltpu.sync_copy(
            acc.at[pl.ds(s * ROWS_PER_TILE, ROWS_PER_TILE)],
            out.at[c, pl.ds(s * ROWS_PER_TILE, ROWS_PER_TILE)],
        )

        @pl.when(s == 0)
        def _():
            pltpu.sync_copy(
                acc.at[pl.ds(NS * ROWS_PER_TILE, ROWS_REM)],
                out.at[c, pl.ds(NS * ROWS_PER_TILE, ROWS_REM)],
            )

    return agg


_agg128 = _make_agg(NHID)


def _stage_a(x, w0, w1k):
    def body(x_ref, w0_ref, w1_ref, h0_ref, s1_ref):
        h0 = jnp.dot(x_ref[...], w0_ref[...], preferred_element_type=jnp.float32)
        h0_ref[...] = h0
        s1_ref[...] = jnp.dot(h0, w1_ref[...], preferred_element_type=jnp.float32)

    return pl.pallas_call(
        body,
        grid=(N // BLK,),
        in_specs=[
            pl.BlockSpec((BLK, NFEAT), lambda i: (i, 0)),
            pl.BlockSpec((NFEAT, NHID), lambda i: (0, 0)),
            pl.BlockSpec((NHID, NHID), lambda i: (0, 0)),
        ],
        out_specs=[
            pl.BlockSpec((BLK, NHID), lambda i: (i, 0)),
            pl.BlockSpec((BLK, NHID), lambda i: (i, 0)),
        ],
        out_shape=[
            jax.ShapeDtypeStruct((N, NHID), jnp.float32),
            jax.ShapeDtypeStruct((N, NHID), jnp.float32),
        ],
    )(x, w0, w1k)


def _stage_b(p, bk, wk):
    def body(p_ref, b_ref, w_ref, s_ref):
        h = jnp.maximum(p_ref[0] + p_ref[1] + b_ref[...], 0.0)
        s_ref[...] = jnp.dot(h, w_ref[...], preferred_element_type=jnp.float32)

    return pl.pallas_call(
        body,
        grid=(N // BLK,),
        in_specs=[
            pl.BlockSpec((NC, BLK, NHID), lambda i: (0, i, 0)),
            pl.BlockSpec((1, NHID), lambda i: (0, 0)),
            pl.BlockSpec((NHID, NHID), lambda i: (0, 0)),
        ],
        out_specs=pl.BlockSpec((BLK, NHID), lambda i: (i, 0)),
        out_shape=jax.ShapeDtypeStruct((N, NHID), jnp.float32),
    )(p, bk, wk)


def _stage_c(p, bk, h0):
    def body(p_ref, b_ref, h0_ref, s_ref):
        h2 = jnp.maximum(p_ref[0] + p_ref[1] + b_ref[...], 0.0)
        s_ref[...] = h2 + h0_ref[...]

    return pl.pallas_call(
        body,
        grid=(N // BLK,),
        in_specs=[
            pl.BlockSpec((NC, BLK, NHID), lambda i: (0, i, 0)),
            pl.BlockSpec((1, NHID), lambda i: (0, 0)),
            pl.BlockSpec((BLK, NHID), lambda i: (i, 0)),
        ],
        out_specs=pl.BlockSpec((BLK, NHID), lambda i: (i, 0)),
        out_shape=jax.ShapeDtypeStruct((N, NHID), jnp.float32),
    )(p, bk, h0)


def _stage_d(p, w2k, b2k):
    def body(p_ref, w_ref, b_ref, o_ref):
        agg = p_ref[0] + p_ref[1]
        o_ref[...] = jnp.dot(agg, w_ref[...],
                             preferred_element_type=jnp.float32) + b_ref[...]

    return pl.pallas_call(
        body,
        grid=(N // BLK,),
        in_specs=[
            pl.BlockSpec((NC, BLK, NHID), lambda i: (0, i, 0)),
            pl.BlockSpec((NHID, NCLASS), lambda i: (0, 0)),
            pl.BlockSpec((1, NCLASS), lambda i: (0, 0)),
        ],
        out_specs=pl.BlockSpec((BLK, NCLASS), lambda i: (i, 0)),
        out_shape=jax.ShapeDtypeStruct((N, NCLASS), jnp.float32),
    )(p, w2k, b2k)


def kernel(x, edge_index, W0, W1, b1, Wm, bm, W2, b2):
    kk = jnp.float32(1.0) / jnp.sqrt(jnp.float32(NHID))
    src3 = edge_index[0].reshape(NW, NCH, CH)
    dst3 = edge_index[1].reshape(NW, NCH, CH)
    zeros128 = jnp.zeros((ROWS_PER_TILE, NHID), jnp.float32)

    h0, s1 = _stage_a(x, W0, W1 * kk)
    p1 = _agg128(s1, src3, dst3, zeros128)
    s2 = _stage_b(p1, (b1 * kk).reshape(1, NHID), Wm * kk)
    p2 = _agg128(s2, src3, dst3, zeros128)
    s3 = _stage_c(p2, (bm * kk).reshape(1, NHID), h0)
    p3 = _agg128(s3, src3, dst3, zeros128)
    out = _stage_d(p3, W2 * kk, (b2 * kk).reshape(1, NCLASS))
    return out
